# Initial kernel scaffold; baseline (speedup 1.0000x reference)
import jax
import jax.numpy as jnp
from jax.experimental import pallas as pl


def kernel(x0, pos0, batch0, lin_w, lin_b, pred_w, pred_b):
    raise NotImplementedError("write your pallas kernel here")


# speedup vs baseline: 1.1723x; 1.1723x over previous
import functools

import jax
import jax.numpy as jnp
from jax.experimental import pallas as pl

N = 10000
C = 64
RADIUS = 0.25
K32 = 32
K10 = 10


def _finalize_body(cg_ref, Rg_ref, tg_ref, pos_ref, xo_ref, Ro_ref, to_ref):
    cg = cg_ref[...]
    Rg = Rg_ref[...]
    tg = tg_ref[...]
    pos = pos_ref[...]
    den = jnp.sum(cg, axis=1)[:, None]
    Rnum = jnp.zeros((cg.shape[0], 9), jnp.float32)
    tnum = jnp.zeros((cg.shape[0], 3), jnp.float32)
    for j in range(K10):
        cj = cg[:, j:j + 1]
        Rnum = Rnum + cj * Rg[:, 9 * j:9 * j + 9]
        tnum = tnum + cj * tg[:, 3 * j:3 * j + 3]
    Ro = Rnum / den
    to = tnum / den
    xo = jnp.stack(
        [jnp.sum(Ro[:, 3 * i:3 * i + 3] * pos, axis=1) for i in range(3)],
        axis=1) + to
    xo_ref[...] = xo
    Ro_ref[...] = Ro
    to_ref[...] = to


def _finalize(cg, Rg, tg, pos0):
    B = 2000
    grid = (N // B,)
    return pl.pallas_call(
        _finalize_body,
        grid=grid,
        in_specs=[
            pl.BlockSpec((B, K10), lambda i: (i, 0)),
            pl.BlockSpec((B, 9 * K10), lambda i: (i, 0)),
            pl.BlockSpec((B, 3 * K10), lambda i: (i, 0)),
            pl.BlockSpec((B, 3), lambda i: (i, 0)),
        ],
        out_specs=[
            pl.BlockSpec((B, 3), lambda i: (i, 0)),
            pl.BlockSpec((B, 9), lambda i: (i, 0)),
            pl.BlockSpec((B, 3), lambda i: (i, 0)),
        ],
        out_shape=[
            jax.ShapeDtypeStruct((N, 3), jnp.float32),
            jax.ShapeDtypeStruct((N, 9), jnp.float32),
            jax.ShapeDtypeStruct((N, 3), jnp.float32),
        ],
    )(cg, Rg, tg, pos0)


def _knn32(pos):
    def f(q):
        d = jnp.sum((q[:, None, :] - pos[None, :, :]) ** 2, axis=-1)
        negd, idx = jax.lax.top_k(-d, K32)
        return idx, -negd
    qs = pos.reshape(N // 1000, 1000, 3)
    idx, dist = jax.lax.map(f, qs)
    return idx.reshape(N, K32), dist.reshape(N, K32)


def _fps_select(pos6, m):
    n = pos6.shape[0]
    sel = jnp.zeros((m,), dtype=jnp.int32)
    dist = jnp.full((n,), jnp.inf, dtype=pos6.dtype)
    def body(i, state):
        dist, sel = state
        d = jnp.sum((pos6 - pos6[sel[i]]) ** 2, axis=-1)
        dist = jnp.minimum(dist, d)
        sel = sel.at[i + 1].set(jnp.argmax(dist).astype(jnp.int32))
        return (dist, sel)
    dist, sel = jax.lax.fori_loop(0, m - 1, body, (dist, sel))
    return sel


def _rigid_fitting(pos, x, nbr):
    P = pos[nbr]
    Q = x[nbr]
    pc = P.mean(axis=1, keepdims=True)
    qc = Q.mean(axis=1, keepdims=True)
    H = jnp.einsum('nki,nkj->nij', P - pc, Q - qc)
    U, S, Vt = jnp.linalg.svd(H, full_matrices=False)
    V = jnp.swapaxes(Vt, -1, -2)
    det = jnp.linalg.det(jnp.einsum('nij,nkj->nik', V, U))
    Vc = jnp.concatenate([V[:, :, :2], V[:, :, 2:] * det[:, None, None]], axis=-1)
    R = jnp.einsum('nij,nkj->nik', Vc, U)
    t = qc[:, 0, :] - jnp.einsum('nij,nj->ni', R, pc[:, 0, :])
    return R, t


def kernel(x0, pos0, batch0, lin_w, lin_b, pred_w, pred_b):
    n = N
    ar = jnp.arange(n)

    nbr32, d32 = _knn32(pos0)
    nbr10 = nbr32[:, :K10]

    R0, t0 = _rigid_fitting(pos0, x0, nbr10)

    pos6d = jnp.concatenate([pos0, x0], axis=-1)
    sel = _fps_select(pos6d, C)
    cent = pos6d[sel]
    dcl = jnp.sum((pos6d[:, None, :] - cent[None, :, :]) ** 2, axis=-1)
    cl = jnp.argmin(dcl, axis=1).astype(jnp.int32)

    srcr = jnp.where(d32 <= RADIUS ** 2, nbr32, ar[:, None].astype(nbr32.dtype))

    posj = pos0[srcr]
    xj = x0[srcr]
    Ri = R0[:, None, :, :]
    ti = t0[:, None, :]
    pred = jnp.einsum('neij,nej->nei', jnp.broadcast_to(Ri, (n, K32, 3, 3)), posj) + ti
    dists = jnp.sum((pred - xj) ** 2, axis=-1)
    wts = jax.nn.sigmoid(dists[:, :, None] * lin_w[None, None, :]
                         + lin_b[None, None, :])

    emb = jax.nn.one_hot(cl, C, dtype=x0.dtype)
    def prop(feat, k):
        g = feat[srcr]
        return jnp.max(wts[:, :, k:k+1] * g, axis=1)
    msg1 = prop(emb, 0)
    msg2 = prop(msg1, 1)
    g2 = msg2[srcr]
    msg3 = jnp.max(wts[:, :, 2:3] * g2, axis=1)
    msg4 = jnp.max(wts[:, :, 3:4] * g2, axis=1)
    msg5 = jnp.max(wts[:, :, 4:5] * g2, axis=1)

    msg = jnp.stack([msg1.sum(-1), msg2.sum(-1), msg3.sum(-1),
                     msg4.sum(-1), msg5.sum(-1)], axis=-1)
    conf0 = jax.nn.sigmoid(msg @ pred_w.T + pred_b)[:, 0:1]

    cg = conf0[nbr10, 0]
    Rg = R0.reshape(n, 9)[nbr10].reshape(n, 9 * K10)
    tg = t0[nbr10].reshape(n, 3 * K10)
    x_out, R_out9, t_out = _finalize(cg, Rg, tg, pos0)
    return (x_out, R_out9.reshape(n, 3, 3), t_out)


# speedup vs baseline: 7.9131x; 6.7500x over previous
import functools

import jax
import jax.numpy as jnp
from jax.experimental import pallas as pl

N = 10000
C = 64
RADIUS = 0.25
K32 = 32
K10 = 10


def _finalize_body(cg_ref, Rg_ref, tg_ref, pos_ref, xo_ref, Ro_ref, to_ref):
    cg = cg_ref[...]
    Rg = Rg_ref[...]
    tg = tg_ref[...]
    pos = pos_ref[...]
    den = jnp.sum(cg, axis=1)[:, None]
    Rnum = jnp.zeros((cg.shape[0], 9), jnp.float32)
    tnum = jnp.zeros((cg.shape[0], 3), jnp.float32)
    for j in range(K10):
        cj = cg[:, j:j + 1]
        Rnum = Rnum + cj * Rg[:, 9 * j:9 * j + 9]
        tnum = tnum + cj * tg[:, 3 * j:3 * j + 3]
    Ro = Rnum / den
    to = tnum / den
    xo = jnp.stack(
        [jnp.sum(Ro[:, 3 * i:3 * i + 3] * pos, axis=1) for i in range(3)],
        axis=1) + to
    xo_ref[...] = xo
    Ro_ref[...] = Ro
    to_ref[...] = to


def _finalize(cg, Rg, tg, pos0):
    B = 2000
    grid = (N // B,)
    return pl.pallas_call(
        _finalize_body,
        grid=grid,
        in_specs=[
            pl.BlockSpec((B, K10), lambda i: (i, 0)),
            pl.BlockSpec((B, 9 * K10), lambda i: (i, 0)),
            pl.BlockSpec((B, 3 * K10), lambda i: (i, 0)),
            pl.BlockSpec((B, 3), lambda i: (i, 0)),
        ],
        out_specs=[
            pl.BlockSpec((B, 3), lambda i: (i, 0)),
            pl.BlockSpec((B, 9), lambda i: (i, 0)),
            pl.BlockSpec((B, 3), lambda i: (i, 0)),
        ],
        out_shape=[
            jax.ShapeDtypeStruct((N, 3), jnp.float32),
            jax.ShapeDtypeStruct((N, 9), jnp.float32),
            jax.ShapeDtypeStruct((N, 3), jnp.float32),
        ],
    )(cg, Rg, tg, pos0)


_BCOLS = 10112
_QROWS = 200


def _knn_body(q_ref, b_ref, idx_ref, dv_ref, d_ref):
    from jax.experimental.pallas import tpu as pltpu
    q = q_ref[...]
    qx, qy, qz = q[:, 0:1], q[:, 1:2], q[:, 2:3]
    bx = b_ref[0:1, :]
    by = b_ref[1:2, :]
    bz = b_ref[2:3, :]
    d_ref[...] = (qx - bx) ** 2 + (qy - by) ** 2 + (qz - bz) ** 2
    col = jax.lax.broadcasted_iota(jnp.int32, (_QROWS, _BCOLS), 1)
    lane = jax.lax.broadcasted_iota(jnp.int32, (_QROWS, K32), 1)

    def body(t, acc):
        acc_idx, acc_dv = acc
        d = d_ref[...]
        dmin = jnp.min(d, axis=1, keepdims=True)
        eq = d == dmin
        idx = jnp.min(jnp.where(eq, col, jnp.int32(2 ** 30)),
                      axis=1, keepdims=True)
        d_ref[...] = jnp.where(col == idx, jnp.float32(3e38), d)
        acc_idx = jnp.where(lane == t, idx, acc_idx)
        acc_dv = jnp.where(lane == t, dmin, acc_dv)
        return acc_idx, acc_dv

    acc_idx = jnp.zeros((_QROWS, K32), jnp.int32)
    acc_dv = jnp.zeros((_QROWS, K32), jnp.float32)
    acc_idx, acc_dv = jax.lax.fori_loop(0, K32, body, (acc_idx, acc_dv))
    idx_ref[...] = acc_idx
    dv_ref[...] = acc_dv


def _knn32(pos):
    from jax.experimental.pallas import tpu as pltpu
    posT = jnp.pad(pos, ((0, _BCOLS - N), (0, 0)),
                   constant_values=1e4).T
    idx, dv = pl.pallas_call(
        _knn_body,
        grid=(N // _QROWS,),
        in_specs=[
            pl.BlockSpec((_QROWS, 3), lambda i: (i, 0)),
            pl.BlockSpec((3, _BCOLS), lambda i: (0, 0)),
        ],
        out_specs=[
            pl.BlockSpec((_QROWS, K32), lambda i: (i, 0)),
            pl.BlockSpec((_QROWS, K32), lambda i: (i, 0)),
        ],
        out_shape=[
            jax.ShapeDtypeStruct((N, K32), jnp.int32),
            jax.ShapeDtypeStruct((N, K32), jnp.float32),
        ],
        scratch_shapes=[pltpu.VMEM((_QROWS, _BCOLS), jnp.float32)],
    )(pos, posT)
    return idx, dv


def _fps_select(pos6, m):
    n = pos6.shape[0]
    sel = jnp.zeros((m,), dtype=jnp.int32)
    dist = jnp.full((n,), jnp.inf, dtype=pos6.dtype)
    def body(i, state):
        dist, sel = state
        d = jnp.sum((pos6 - pos6[sel[i]]) ** 2, axis=-1)
        dist = jnp.minimum(dist, d)
        sel = sel.at[i + 1].set(jnp.argmax(dist).astype(jnp.int32))
        return (dist, sel)
    dist, sel = jax.lax.fori_loop(0, m - 1, body, (dist, sel))
    return sel


_NPAD = 10240
_RT = _NPAD // 128


def _rigid_body(Pt_ref, Qt_ref, R_ref, t_ref):
    P = [Pt_ref[j] for j in range(30)]
    Q = [Qt_ref[j] for j in range(30)]
    pc = [sum(P[3 * k + i] for k in range(K10)) * (1.0 / K10) for i in range(3)]
    qc = [sum(Q[3 * k + i] for k in range(K10)) * (1.0 / K10) for i in range(3)]
    S = [[None] * 3 for _ in range(3)]
    for i in range(3):
        for j in range(3):
            acc = None
            for k in range(K10):
                term = (P[3 * k + i] - pc[i]) * (Q[3 * k + j] - qc[j])
                acc = term if acc is None else acc + term
            S[i][j] = acc
    Sxx, Sxy, Sxz = S[0]
    Syx, Syy, Syz = S[1]
    Szx, Szy, Szz = S[2]
    b00 = Sxx + Syy + Szz
    b01 = Syz - Szy
    b02 = Szx - Sxz
    b03 = Sxy - Syx
    b11 = Sxx - Syy - Szz
    b12 = Sxy + Syx
    b13 = Szx + Sxz
    b22 = -Sxx + Syy - Szz
    b23 = Syz + Szy
    b33 = -Sxx - Syy + Szz
    fro = jnp.sqrt(b00 * b00 + b11 * b11 + b22 * b22 + b33 * b33
                   + 2.0 * (b01 * b01 + b02 * b02 + b03 * b03
                            + b12 * b12 + b13 * b13 + b23 * b23)) + 1e-30
    b00 = b00 + fro
    b11 = b11 + fro
    b22 = b22 + fro
    b33 = b33 + fro
    for _ in range(24):
        n00 = b00 * b00 + b01 * b01 + b02 * b02 + b03 * b03
        n01 = b00 * b01 + b01 * b11 + b02 * b12 + b03 * b13
        n02 = b00 * b02 + b01 * b12 + b02 * b22 + b03 * b23
        n03 = b00 * b03 + b01 * b13 + b02 * b23 + b03 * b33
        n11 = b01 * b01 + b11 * b11 + b12 * b12 + b13 * b13
        n12 = b01 * b02 + b11 * b12 + b12 * b22 + b13 * b23
        n13 = b01 * b03 + b11 * b13 + b12 * b23 + b13 * b33
        n22 = b02 * b02 + b12 * b12 + b22 * b22 + b23 * b23
        n23 = b02 * b03 + b12 * b13 + b22 * b23 + b23 * b33
        n33 = b03 * b03 + b13 * b13 + b23 * b23 + b33 * b33
        nrm = jax.lax.rsqrt(n00 * n00 + n11 * n11 + n22 * n22 + n33 * n33
                            + 2.0 * (n01 * n01 + n02 * n02 + n03 * n03
                                     + n12 * n12 + n13 * n13 + n23 * n23)
                            + 1e-38)
        b00 = n00 * nrm
        b01 = n01 * nrm
        b02 = n02 * nrm
        b03 = n03 * nrm
        b11 = n11 * nrm
        b12 = n12 * nrm
        b13 = n13 * nrm
        b22 = n22 * nrm
        b23 = n23 * nrm
        b33 = n33 * nrm
    c0 = jnp.logical_and(jnp.logical_and(b00 >= b11, b00 >= b22), b00 >= b33)
    c1 = jnp.logical_and(b11 >= b22, b11 >= b33)
    c2 = b22 >= b33
    def pick(v0, v1, v2, v3):
        return jnp.where(c0, v0, jnp.where(c1, v1, jnp.where(c2, v2, v3)))
    qw = pick(b00, b01, b02, b03)
    qx = pick(b01, b11, b12, b13)
    qy = pick(b02, b12, b22, b23)
    qz = pick(b03, b13, b23, b33)
    qn = jax.lax.rsqrt(qw * qw + qx * qx + qy * qy + qz * qz + 1e-38)
    qw, qx, qy, qz = qw * qn, qx * qn, qy * qn, qz * qn
    r00 = 1.0 - 2.0 * (qy * qy + qz * qz)
    r01 = 2.0 * (qx * qy - qw * qz)
    r02 = 2.0 * (qx * qz + qw * qy)
    r10 = 2.0 * (qx * qy + qw * qz)
    r11 = 1.0 - 2.0 * (qx * qx + qz * qz)
    r12 = 2.0 * (qy * qz - qw * qx)
    r20 = 2.0 * (qx * qz - qw * qy)
    r21 = 2.0 * (qy * qz + qw * qx)
    r22 = 1.0 - 2.0 * (qx * qx + qy * qy)
    R = [r00, r01, r02, r10, r11, r12, r20, r21, r22]
    for j in range(9):
        R_ref[j] = R[j]
    t_ref[0] = qc[0] - (r00 * pc[0] + r01 * pc[1] + r02 * pc[2])
    t_ref[1] = qc[1] - (r10 * pc[0] + r11 * pc[1] + r12 * pc[2])
    t_ref[2] = qc[2] - (r20 * pc[0] + r21 * pc[1] + r22 * pc[2])


def _rigid_fitting(pos, x, nbr):
    P = pos[nbr].reshape(N, 30)
    Q = x[nbr].reshape(N, 30)
    pad = ((0, _NPAD - N), (0, 0))
    Pt = jnp.pad(P, pad).T.reshape(30, _RT, 128)
    Qt = jnp.pad(Q, pad).T.reshape(30, _RT, 128)
    R9, t3 = pl.pallas_call(
        _rigid_body,
        in_specs=[
            pl.BlockSpec((30, _RT, 128), lambda: (0, 0, 0)),
            pl.BlockSpec((30, _RT, 128), lambda: (0, 0, 0)),
        ],
        out_specs=[
            pl.BlockSpec((9, _RT, 128), lambda: (0, 0, 0)),
            pl.BlockSpec((3, _RT, 128), lambda: (0, 0, 0)),
        ],
        out_shape=[
            jax.ShapeDtypeStruct((9, _RT, 128), jnp.float32),
            jax.ShapeDtypeStruct((3, _RT, 128), jnp.float32),
        ],
    )(Pt, Qt)
    R = R9.reshape(9, _NPAD).T[:N].reshape(N, 3, 3)
    t = t3.reshape(3, _NPAD).T[:N]
    return R, t


def kernel(x0, pos0, batch0, lin_w, lin_b, pred_w, pred_b):
    n = N
    ar = jnp.arange(n)

    nbr32, d32 = _knn32(pos0)
    nbr10 = nbr32[:, :K10]

    R0, t0 = _rigid_fitting(pos0, x0, nbr10)

    pos6d = jnp.concatenate([pos0, x0], axis=-1)
    sel = _fps_select(pos6d, C)
    cent = pos6d[sel]
    dcl = jnp.sum((pos6d[:, None, :] - cent[None, :, :]) ** 2, axis=-1)
    cl = jnp.argmin(dcl, axis=1).astype(jnp.int32)

    srcr = jnp.where(d32 <= RADIUS ** 2, nbr32, ar[:, None].astype(nbr32.dtype))

    posj = pos0[srcr]
    xj = x0[srcr]
    Ri = R0[:, None, :, :]
    ti = t0[:, None, :]
    pred = jnp.einsum('neij,nej->nei', jnp.broadcast_to(Ri, (n, K32, 3, 3)), posj) + ti
    dists = jnp.sum((pred - xj) ** 2, axis=-1)
    wts = jax.nn.sigmoid(dists[:, :, None] * lin_w[None, None, :]
                         + lin_b[None, None, :])

    emb = jax.nn.one_hot(cl, C, dtype=x0.dtype)
    def prop(feat, k):
        g = feat[srcr]
        return jnp.max(wts[:, :, k:k+1] * g, axis=1)
    msg1 = prop(emb, 0)
    msg2 = prop(msg1, 1)
    g2 = msg2[srcr]
    msg3 = jnp.max(wts[:, :, 2:3] * g2, axis=1)
    msg4 = jnp.max(wts[:, :, 3:4] * g2, axis=1)
    msg5 = jnp.max(wts[:, :, 4:5] * g2, axis=1)

    msg = jnp.stack([msg1.sum(-1), msg2.sum(-1), msg3.sum(-1),
                     msg4.sum(-1), msg5.sum(-1)], axis=-1)
    conf0 = jax.nn.sigmoid(msg @ pred_w.T + pred_b)[:, 0:1]

    cg = conf0[nbr10, 0]
    Rg = R0.reshape(n, 9)[nbr10].reshape(n, 9 * K10)
    tg = t0[nbr10].reshape(n, 3 * K10)
    x_out, R_out9, t_out = _finalize(cg, Rg, tg, pos0)
    return (x_out, R_out9.reshape(n, 3, 3), t_out)


# speedup vs baseline: 11.5920x; 1.4649x over previous
import functools

import jax
import jax.numpy as jnp
from jax.experimental import pallas as pl
from jax.experimental.pallas import tpu as pltpu
from jax.experimental.pallas import tpu_sc as plsc

N = 10000
C = 64
RADIUS = 0.25
K32 = 32
K10 = 10


def _sc_gather(table, idx, window):
    E = idx.shape[0]
    V = table.shape[1]
    idx2 = idx.reshape(1, E)
    mesh = plsc.VectorSubcoreMesh(core_axis_name="c", subcore_axis_name="s")

    @functools.partial(
        pl.kernel,
        out_type=jax.ShapeDtypeStruct((E, V), table.dtype),
        mesh=mesh,
        compiler_params=pltpu.CompilerParams(use_tc_tiling_on_sc=False),
    )
    def gather_kernel(tab_hbm, i_hbm, o_hbm):
        def body(i_vmem, o_vmem):
            pltpu.sync_copy(tab_hbm.at[i_vmem.at[0]], o_vmem)

        pltpu.emit_pipeline(
            body,
            grid=(E // window,),
            in_specs=[pl.BlockSpec((1, window), lambda i: (0, i))],
            out_specs=[pl.BlockSpec((window, V), lambda i: (i, 0))],
            core_axis_name=("c", "s"),
            dimension_semantics=(pltpu.PARALLEL,),
        )(i_hbm, o_hbm)

    return gather_kernel(table, idx2)


def _finalize_body(G_ref, pos_ref, xo_ref, Ro_ref, to_ref):
    G = G_ref[...]
    pos = pos_ref[...]
    den = jnp.zeros((G.shape[0], 1), jnp.float32)
    Rnum = jnp.zeros((G.shape[0], 9), jnp.float32)
    tnum = jnp.zeros((G.shape[0], 3), jnp.float32)
    for j in range(K10):
        cj = G[:, 16 * j:16 * j + 1]
        den = den + cj
        Rnum = Rnum + cj * G[:, 16 * j + 1:16 * j + 10]
        tnum = tnum + cj * G[:, 16 * j + 10:16 * j + 13]
    Ro = Rnum / den
    to = tnum / den
    xo = jnp.stack(
        [jnp.sum(Ro[:, 3 * i:3 * i + 3] * pos, axis=1) for i in range(3)],
        axis=1) + to
    xo_ref[...] = xo
    Ro_ref[...] = Ro
    to_ref[...] = to


def _finalize(G4, pos0):
    B = 2000
    grid = (N // B,)
    return pl.pallas_call(
        _finalize_body,
        grid=grid,
        in_specs=[
            pl.BlockSpec((B, 16 * K10), lambda i: (i, 0)),
            pl.BlockSpec((B, 3), lambda i: (i, 0)),
        ],
        out_specs=[
            pl.BlockSpec((B, 3), lambda i: (i, 0)),
            pl.BlockSpec((B, 9), lambda i: (i, 0)),
            pl.BlockSpec((B, 3), lambda i: (i, 0)),
        ],
        out_shape=[
            jax.ShapeDtypeStruct((N, 3), jnp.float32),
            jax.ShapeDtypeStruct((N, 9), jnp.float32),
            jax.ShapeDtypeStruct((N, 3), jnp.float32),
        ],
    )(G4, pos0)


_MB = 400


def _edge_body(G_ref, R_ref, t_ref, p_ref, d_ref, m1_ref):
    G = G_ref[...]
    r = [R_ref[:, j:j + 1] for j in range(9)]
    tc = [t_ref[:, j:j + 1] for j in range(3)]
    lw0 = p_ref[0:1, 0:1]
    lb0 = p_ref[1:2, 0:1]
    lane64 = jax.lax.broadcasted_iota(jnp.int32, (_MB, C), 1).astype(jnp.float32)
    lane32 = jax.lax.broadcasted_iota(jnp.int32, (_MB, K32), 1)
    msg1 = jnp.zeros((_MB, C), jnp.float32)
    dacc = jnp.zeros((_MB, K32), jnp.float32)
    for e in range(K32):
        b = 16 * e
        px, py, pz = G[:, b:b + 1], G[:, b + 1:b + 2], G[:, b + 2:b + 3]
        xx, xy, xz = G[:, b + 3:b + 4], G[:, b + 4:b + 5], G[:, b + 5:b + 6]
        clv = G[:, b + 6:b + 7]
        e0 = r[0] * px + r[1] * py + r[2] * pz + tc[0] - xx
        e1 = r[3] * px + r[4] * py + r[5] * pz + tc[1] - xy
        e2 = r[6] * px + r[7] * py + r[8] * pz + tc[2] - xz
        de = e0 * e0 + e1 * e1 + e2 * e2
        w0 = jax.nn.sigmoid(de * lw0 + lb0)
        msg1 = jnp.maximum(msg1, jnp.where(clv == lane64, w0, 0.0))
        dacc = jnp.where(lane32 == e, de, dacc)
    d_ref[...] = dacc
    m1_ref[...] = msg1


def _edge_stage(G1, R9, t0, params):
    return pl.pallas_call(
        _edge_body,
        grid=(N // _MB,),
        in_specs=[
            pl.BlockSpec((_MB, 16 * K32), lambda i: (i, 0)),
            pl.BlockSpec((_MB, 9), lambda i: (i, 0)),
            pl.BlockSpec((_MB, 3), lambda i: (i, 0)),
            pl.BlockSpec((8, 128), lambda i: (0, 0)),
        ],
        out_specs=[
            pl.BlockSpec((_MB, K32), lambda i: (i, 0)),
            pl.BlockSpec((_MB, C), lambda i: (i, 0)),
        ],
        out_shape=[
            jax.ShapeDtypeStruct((N, K32), jnp.float32),
            jax.ShapeDtypeStruct((N, C), jnp.float32),
        ],
    )(G1, R9, t0, params)


def _prop_max(G, d, p_ref, k_list):
    outs = [jnp.full((_MB, C), -jnp.inf, jnp.float32) for _ in k_list]
    for e in range(K32):
        de = d[:, e:e + 1]
        g = G[:, C * e:C * e + C]
        for s, k in enumerate(k_list):
            w = jax.nn.sigmoid(de * p_ref[0:1, k:k + 1] + p_ref[1:2, k:k + 1])
            outs[s] = jnp.maximum(outs[s], w * g)
    return outs


def _prop2_body(G_ref, d_ref, p_ref, m2_ref):
    (m2,) = _prop_max(G_ref[...], d_ref[...], p_ref, [1])
    m2_ref[...] = m2


def _prop345_conf_body(G_ref, d_ref, m1_ref, m2_ref, p_ref, conf_ref):
    m345 = _prop_max(G_ref[...], d_ref[...], p_ref, [2, 3, 4])
    s1 = jnp.sum(m1_ref[...], axis=1, keepdims=True)
    s2 = jnp.sum(m2_ref[...], axis=1, keepdims=True)
    s3 = jnp.sum(m345[0], axis=1, keepdims=True)
    s4 = jnp.sum(m345[1], axis=1, keepdims=True)
    s5 = jnp.sum(m345[2], axis=1, keepdims=True)
    p = p_ref
    logit = (s1 * p[2:3, 0:1] + s2 * p[2:3, 1:2] + s3 * p[2:3, 2:3]
             + s4 * p[2:3, 3:4] + s5 * p[2:3, 4:5] + p[3:4, 0:1])
    conf_ref[...] = jax.nn.sigmoid(logit)


def _prop2(G2, dists, params):
    return pl.pallas_call(
        _prop2_body,
        grid=(N // _MB,),
        in_specs=[
            pl.BlockSpec((_MB, C * K32), lambda i: (i, 0)),
            pl.BlockSpec((_MB, K32), lambda i: (i, 0)),
            pl.BlockSpec((8, 128), lambda i: (0, 0)),
        ],
        out_specs=pl.BlockSpec((_MB, C), lambda i: (i, 0)),
        out_shape=jax.ShapeDtypeStruct((N, C), jnp.float32),
    )(G2, dists, params)


def _prop345_conf(G3, dists, msg1, msg2, params):
    return pl.pallas_call(
        _prop345_conf_body,
        grid=(N // _MB,),
        in_specs=[
            pl.BlockSpec((_MB, C * K32), lambda i: (i, 0)),
            pl.BlockSpec((_MB, K32), lambda i: (i, 0)),
            pl.BlockSpec((_MB, C), lambda i: (i, 0)),
            pl.BlockSpec((_MB, C), lambda i: (i, 0)),
            pl.BlockSpec((8, 128), lambda i: (0, 0)),
        ],
        out_specs=pl.BlockSpec((_MB, 1), lambda i: (i, 0)),
        out_shape=jax.ShapeDtypeStruct((N, 1), jnp.float32),
    )(G3, dists, msg1, msg2, params)


_BCOLS = 10112
_QROWS = 200


def _knn_body(q_ref, b_ref, idx_ref, dv_ref, d_ref):
    from jax.experimental.pallas import tpu as pltpu
    q = q_ref[...]
    qx, qy, qz = q[:, 0:1], q[:, 1:2], q[:, 2:3]
    bx = b_ref[0:1, :]
    by = b_ref[1:2, :]
    bz = b_ref[2:3, :]
    d_ref[...] = (qx - bx) ** 2 + (qy - by) ** 2 + (qz - bz) ** 2
    col = jax.lax.broadcasted_iota(jnp.int32, (_QROWS, _BCOLS), 1)
    lane = jax.lax.broadcasted_iota(jnp.int32, (_QROWS, K32), 1)

    def body(t, acc):
        acc_idx, acc_dv = acc
        d = d_ref[...]
        dmin = jnp.min(d, axis=1, keepdims=True)
        eq = d == dmin
        idx = jnp.min(jnp.where(eq, col, jnp.int32(2 ** 30)),
                      axis=1, keepdims=True)
        d_ref[...] = jnp.where(col == idx, jnp.float32(3e38), d)
        acc_idx = jnp.where(lane == t, idx, acc_idx)
        acc_dv = jnp.where(lane == t, dmin, acc_dv)
        return acc_idx, acc_dv

    acc_idx = jnp.zeros((_QROWS, K32), jnp.int32)
    acc_dv = jnp.zeros((_QROWS, K32), jnp.float32)
    acc_idx, acc_dv = jax.lax.fori_loop(0, K32, body, (acc_idx, acc_dv))
    idx_ref[...] = acc_idx
    dv_ref[...] = acc_dv


def _knn32(pos):
    from jax.experimental.pallas import tpu as pltpu
    posT = jnp.pad(pos, ((0, _BCOLS - N), (0, 0)),
                   constant_values=1e4).T
    idx, dv = pl.pallas_call(
        _knn_body,
        grid=(N // _QROWS,),
        in_specs=[
            pl.BlockSpec((_QROWS, 3), lambda i: (i, 0)),
            pl.BlockSpec((3, _BCOLS), lambda i: (0, 0)),
        ],
        out_specs=[
            pl.BlockSpec((_QROWS, K32), lambda i: (i, 0)),
            pl.BlockSpec((_QROWS, K32), lambda i: (i, 0)),
        ],
        out_shape=[
            jax.ShapeDtypeStruct((N, K32), jnp.int32),
            jax.ShapeDtypeStruct((N, K32), jnp.float32),
        ],
        scratch_shapes=[pltpu.VMEM((_QROWS, _BCOLS), jnp.float32)],
    )(pos, posT)
    return idx, dv


def _fps_select(pos6, m):
    n = pos6.shape[0]
    sel = jnp.zeros((m,), dtype=jnp.int32)
    dist = jnp.full((n,), jnp.inf, dtype=pos6.dtype)
    def body(i, state):
        dist, sel = state
        d = jnp.sum((pos6 - pos6[sel[i]]) ** 2, axis=-1)
        dist = jnp.minimum(dist, d)
        sel = sel.at[i + 1].set(jnp.argmax(dist).astype(jnp.int32))
        return (dist, sel)
    dist, sel = jax.lax.fori_loop(0, m - 1, body, (dist, sel))
    return sel


_NPAD = 10240
_RT = _NPAD // 128


def _rigid_body(Pt_ref, Qt_ref, R_ref, t_ref):
    P = [Pt_ref[j] for j in range(30)]
    Q = [Qt_ref[j] for j in range(30)]
    pc = [sum(P[3 * k + i] for k in range(K10)) * (1.0 / K10) for i in range(3)]
    qc = [sum(Q[3 * k + i] for k in range(K10)) * (1.0 / K10) for i in range(3)]
    S = [[None] * 3 for _ in range(3)]
    for i in range(3):
        for j in range(3):
            acc = None
            for k in range(K10):
                term = (P[3 * k + i] - pc[i]) * (Q[3 * k + j] - qc[j])
                acc = term if acc is None else acc + term
            S[i][j] = acc
    Sxx, Sxy, Sxz = S[0]
    Syx, Syy, Syz = S[1]
    Szx, Szy, Szz = S[2]
    b00 = Sxx + Syy + Szz
    b01 = Syz - Szy
    b02 = Szx - Sxz
    b03 = Sxy - Syx
    b11 = Sxx - Syy - Szz
    b12 = Sxy + Syx
    b13 = Szx + Sxz
    b22 = -Sxx + Syy - Szz
    b23 = Syz + Szy
    b33 = -Sxx - Syy + Szz
    fro = jnp.sqrt(b00 * b00 + b11 * b11 + b22 * b22 + b33 * b33
                   + 2.0 * (b01 * b01 + b02 * b02 + b03 * b03
                            + b12 * b12 + b13 * b13 + b23 * b23)) + 1e-30
    b00 = b00 + fro
    b11 = b11 + fro
    b22 = b22 + fro
    b33 = b33 + fro
    for _ in range(24):
        n00 = b00 * b00 + b01 * b01 + b02 * b02 + b03 * b03
        n01 = b00 * b01 + b01 * b11 + b02 * b12 + b03 * b13
        n02 = b00 * b02 + b01 * b12 + b02 * b22 + b03 * b23
        n03 = b00 * b03 + b01 * b13 + b02 * b23 + b03 * b33
        n11 = b01 * b01 + b11 * b11 + b12 * b12 + b13 * b13
        n12 = b01 * b02 + b11 * b12 + b12 * b22 + b13 * b23
        n13 = b01 * b03 + b11 * b13 + b12 * b23 + b13 * b33
        n22 = b02 * b02 + b12 * b12 + b22 * b22 + b23 * b23
        n23 = b02 * b03 + b12 * b13 + b22 * b23 + b23 * b33
        n33 = b03 * b03 + b13 * b13 + b23 * b23 + b33 * b33
        nrm = jax.lax.rsqrt(n00 * n00 + n11 * n11 + n22 * n22 + n33 * n33
                            + 2.0 * (n01 * n01 + n02 * n02 + n03 * n03
                                     + n12 * n12 + n13 * n13 + n23 * n23)
                            + 1e-38)
        b00 = n00 * nrm
        b01 = n01 * nrm
        b02 = n02 * nrm
        b03 = n03 * nrm
        b11 = n11 * nrm
        b12 = n12 * nrm
        b13 = n13 * nrm
        b22 = n22 * nrm
        b23 = n23 * nrm
        b33 = n33 * nrm
    c0 = jnp.logical_and(jnp.logical_and(b00 >= b11, b00 >= b22), b00 >= b33)
    c1 = jnp.logical_and(b11 >= b22, b11 >= b33)
    c2 = b22 >= b33
    def pick(v0, v1, v2, v3):
        return jnp.where(c0, v0, jnp.where(c1, v1, jnp.where(c2, v2, v3)))
    qw = pick(b00, b01, b02, b03)
    qx = pick(b01, b11, b12, b13)
    qy = pick(b02, b12, b22, b23)
    qz = pick(b03, b13, b23, b33)
    qn = jax.lax.rsqrt(qw * qw + qx * qx + qy * qy + qz * qz + 1e-38)
    qw, qx, qy, qz = qw * qn, qx * qn, qy * qn, qz * qn
    r00 = 1.0 - 2.0 * (qy * qy + qz * qz)
    r01 = 2.0 * (qx * qy - qw * qz)
    r02 = 2.0 * (qx * qz + qw * qy)
    r10 = 2.0 * (qx * qy + qw * qz)
    r11 = 1.0 - 2.0 * (qx * qx + qz * qz)
    r12 = 2.0 * (qy * qz - qw * qx)
    r20 = 2.0 * (qx * qz - qw * qy)
    r21 = 2.0 * (qy * qz + qw * qx)
    r22 = 1.0 - 2.0 * (qx * qx + qy * qy)
    R = [r00, r01, r02, r10, r11, r12, r20, r21, r22]
    for j in range(9):
        R_ref[j] = R[j]
    t_ref[0] = qc[0] - (r00 * pc[0] + r01 * pc[1] + r02 * pc[2])
    t_ref[1] = qc[1] - (r10 * pc[0] + r11 * pc[1] + r12 * pc[2])
    t_ref[2] = qc[2] - (r20 * pc[0] + r21 * pc[1] + r22 * pc[2])


def _rigid_fitting(pos, x, nbr):
    P = pos[nbr].reshape(N, 30)
    Q = x[nbr].reshape(N, 30)
    pad = ((0, _NPAD - N), (0, 0))
    Pt = jnp.pad(P, pad).T.reshape(30, _RT, 128)
    Qt = jnp.pad(Q, pad).T.reshape(30, _RT, 128)
    R9, t3 = pl.pallas_call(
        _rigid_body,
        in_specs=[
            pl.BlockSpec((30, _RT, 128), lambda: (0, 0, 0)),
            pl.BlockSpec((30, _RT, 128), lambda: (0, 0, 0)),
        ],
        out_specs=[
            pl.BlockSpec((9, _RT, 128), lambda: (0, 0, 0)),
            pl.BlockSpec((3, _RT, 128), lambda: (0, 0, 0)),
        ],
        out_shape=[
            jax.ShapeDtypeStruct((9, _RT, 128), jnp.float32),
            jax.ShapeDtypeStruct((3, _RT, 128), jnp.float32),
        ],
    )(Pt, Qt)
    R = R9.reshape(9, _NPAD).T[:N].reshape(N, 3, 3)
    t = t3.reshape(3, _NPAD).T[:N]
    return R, t


def kernel(x0, pos0, batch0, lin_w, lin_b, pred_w, pred_b):
    n = N
    ar = jnp.arange(n)

    nbr32, d32 = _knn32(pos0)
    nbr10 = nbr32[:, :K10]

    R0, t0 = _rigid_fitting(pos0, x0, nbr10)

    pos6d = jnp.concatenate([pos0, x0], axis=-1)
    sel = _fps_select(pos6d, C)
    cent = pos6d[sel]
    dcl = jnp.sum((pos6d[:, None, :] - cent[None, :, :]) ** 2, axis=-1)
    cl = jnp.argmin(dcl, axis=1).astype(jnp.int32)

    srcr = jnp.where(d32 <= RADIUS ** 2, nbr32, ar[:, None].astype(nbr32.dtype))
    srcf = srcr.reshape(-1)

    params = (jnp.zeros((8, 128), jnp.float32)
              .at[0, :5].set(lin_w)
              .at[1, :5].set(lin_b)
              .at[2, :5].set(pred_w[0])
              .at[3, 0].set(pred_b[0]))
    R9 = R0.reshape(n, 9)

    T1 = jnp.concatenate([pos0, x0, cl[:, None].astype(jnp.float32),
                          jnp.zeros((n, 9), jnp.float32)], axis=1)
    G1 = _sc_gather(T1, srcf, 128).reshape(n, 16 * K32)
    dists, msg1 = _edge_stage(G1, R9, t0, params)

    G2 = _sc_gather(msg1, srcf, 128).reshape(n, C * K32)
    msg2 = _prop2(G2, dists, params)
    G3 = _sc_gather(msg2, srcf, 128).reshape(n, C * K32)
    conf0 = _prop345_conf(G3, dists, msg1, msg2, params)

    T4 = jnp.concatenate([conf0, R9, t0,
                          jnp.zeros((n, 3), jnp.float32)], axis=1)
    G4 = _sc_gather(T4, nbr10.reshape(-1), 160).reshape(n, 16 * K10)
    x_out, R_out9, t_out = _finalize(G4, pos0)
    return (x_out, R_out9.reshape(n, 3, 3), t_out)


# speedup vs baseline: 11.8293x; 1.0205x over previous
import functools

import jax
import jax.numpy as jnp
from jax.experimental import pallas as pl
from jax.experimental.pallas import tpu as pltpu
from jax.experimental.pallas import tpu_sc as plsc

N = 10000
C = 64
RADIUS = 0.25
K32 = 32
K10 = 10


def _sc_gather(table, idx, window):
    E = idx.shape[0]
    V = table.shape[1]
    idx2 = idx.reshape(1, E)
    mesh = plsc.VectorSubcoreMesh(core_axis_name="c", subcore_axis_name="s")

    @functools.partial(
        pl.kernel,
        out_type=jax.ShapeDtypeStruct((E, V), table.dtype),
        mesh=mesh,
        compiler_params=pltpu.CompilerParams(use_tc_tiling_on_sc=False),
    )
    def gather_kernel(tab_hbm, i_hbm, o_hbm):
        def body(i_vmem, o_vmem):
            pltpu.sync_copy(tab_hbm.at[i_vmem.at[0]], o_vmem)

        pltpu.emit_pipeline(
            body,
            grid=(E // window,),
            in_specs=[pl.BlockSpec((1, window), lambda i: (0, i))],
            out_specs=[pl.BlockSpec((window, V), lambda i: (i, 0))],
            core_axis_name=("c", "s"),
            dimension_semantics=(pltpu.PARALLEL,),
        )(i_hbm, o_hbm)

    return gather_kernel(table, idx2)


def _finalize_body(G_ref, pos_ref, xo_ref, Ro_ref, to_ref):
    G = G_ref[...]
    pos = pos_ref[...]
    den = jnp.zeros((G.shape[0], 1), jnp.float32)
    Rnum = jnp.zeros((G.shape[0], 9), jnp.float32)
    tnum = jnp.zeros((G.shape[0], 3), jnp.float32)
    for j in range(K10):
        cj = G[:, 16 * j:16 * j + 1]
        den = den + cj
        Rnum = Rnum + cj * G[:, 16 * j + 1:16 * j + 10]
        tnum = tnum + cj * G[:, 16 * j + 10:16 * j + 13]
    Ro = Rnum / den
    to = tnum / den
    xo = jnp.stack(
        [jnp.sum(Ro[:, 3 * i:3 * i + 3] * pos, axis=1) for i in range(3)],
        axis=1) + to
    xo_ref[...] = xo
    Ro_ref[...] = Ro
    to_ref[...] = to


def _finalize(G4, pos0):
    B = 2000
    grid = (N // B,)
    return pl.pallas_call(
        _finalize_body,
        grid=grid,
        in_specs=[
            pl.BlockSpec((B, 16 * K10), lambda i: (i, 0)),
            pl.BlockSpec((B, 3), lambda i: (i, 0)),
        ],
        out_specs=[
            pl.BlockSpec((B, 3), lambda i: (i, 0)),
            pl.BlockSpec((B, 9), lambda i: (i, 0)),
            pl.BlockSpec((B, 3), lambda i: (i, 0)),
        ],
        out_shape=[
            jax.ShapeDtypeStruct((N, 3), jnp.float32),
            jax.ShapeDtypeStruct((N, 9), jnp.float32),
            jax.ShapeDtypeStruct((N, 3), jnp.float32),
        ],
    )(G4, pos0)


_MB = 400


def _edge_body(G_ref, R_ref, t_ref, p_ref, d_ref, m1_ref):
    G = G_ref[...]
    r = [R_ref[:, j:j + 1] for j in range(9)]
    tc = [t_ref[:, j:j + 1] for j in range(3)]
    lw0 = p_ref[0:1, 0:1]
    lb0 = p_ref[1:2, 0:1]
    lane64 = jax.lax.broadcasted_iota(jnp.int32, (_MB, C), 1).astype(jnp.float32)
    lane32 = jax.lax.broadcasted_iota(jnp.int32, (_MB, K32), 1)
    msg1 = jnp.zeros((_MB, C), jnp.float32)
    dacc = jnp.zeros((_MB, K32), jnp.float32)
    for e in range(K32):
        b = 16 * e
        px, py, pz = G[:, b:b + 1], G[:, b + 1:b + 2], G[:, b + 2:b + 3]
        xx, xy, xz = G[:, b + 3:b + 4], G[:, b + 4:b + 5], G[:, b + 5:b + 6]
        clv = G[:, b + 6:b + 7]
        e0 = r[0] * px + r[1] * py + r[2] * pz + tc[0] - xx
        e1 = r[3] * px + r[4] * py + r[5] * pz + tc[1] - xy
        e2 = r[6] * px + r[7] * py + r[8] * pz + tc[2] - xz
        de = e0 * e0 + e1 * e1 + e2 * e2
        w0 = jax.nn.sigmoid(de * lw0 + lb0)
        msg1 = jnp.maximum(msg1, jnp.where(clv == lane64, w0, 0.0))
        dacc = jnp.where(lane32 == e, de, dacc)
    d_ref[...] = dacc
    m1_ref[...] = msg1


def _edge_stage(G1, R9, t0, params):
    return pl.pallas_call(
        _edge_body,
        grid=(N // _MB,),
        in_specs=[
            pl.BlockSpec((_MB, 16 * K32), lambda i: (i, 0)),
            pl.BlockSpec((_MB, 9), lambda i: (i, 0)),
            pl.BlockSpec((_MB, 3), lambda i: (i, 0)),
            pl.BlockSpec((8, 128), lambda i: (0, 0)),
        ],
        out_specs=[
            pl.BlockSpec((_MB, K32), lambda i: (i, 0)),
            pl.BlockSpec((_MB, C), lambda i: (i, 0)),
        ],
        out_shape=[
            jax.ShapeDtypeStruct((N, K32), jnp.float32),
            jax.ShapeDtypeStruct((N, C), jnp.float32),
        ],
    )(G1, R9, t0, params)


def _prop_max(G, d, p_ref, k_list):
    outs = [jnp.full((_MB, C), -jnp.inf, jnp.float32) for _ in k_list]
    for e in range(K32):
        de = d[:, e:e + 1]
        g = G[:, C * e:C * e + C]
        for s, k in enumerate(k_list):
            w = jax.nn.sigmoid(de * p_ref[0:1, k:k + 1] + p_ref[1:2, k:k + 1])
            outs[s] = jnp.maximum(outs[s], w * g)
    return outs


def _prop2_body(G_ref, d_ref, p_ref, m2_ref):
    (m2,) = _prop_max(G_ref[...], d_ref[...], p_ref, [1])
    m2_ref[...] = m2


def _prop345_conf_body(G_ref, d_ref, m1_ref, m2_ref, p_ref, conf_ref):
    m345 = _prop_max(G_ref[...], d_ref[...], p_ref, [2, 3, 4])
    s1 = jnp.sum(m1_ref[...], axis=1, keepdims=True)
    s2 = jnp.sum(m2_ref[...], axis=1, keepdims=True)
    s3 = jnp.sum(m345[0], axis=1, keepdims=True)
    s4 = jnp.sum(m345[1], axis=1, keepdims=True)
    s5 = jnp.sum(m345[2], axis=1, keepdims=True)
    p = p_ref
    logit = (s1 * p[2:3, 0:1] + s2 * p[2:3, 1:2] + s3 * p[2:3, 2:3]
             + s4 * p[2:3, 3:4] + s5 * p[2:3, 4:5] + p[3:4, 0:1])
    conf_ref[...] = jax.nn.sigmoid(logit)


def _prop2(G2, dists, params):
    return pl.pallas_call(
        _prop2_body,
        grid=(N // _MB,),
        in_specs=[
            pl.BlockSpec((_MB, C * K32), lambda i: (i, 0)),
            pl.BlockSpec((_MB, K32), lambda i: (i, 0)),
            pl.BlockSpec((8, 128), lambda i: (0, 0)),
        ],
        out_specs=pl.BlockSpec((_MB, C), lambda i: (i, 0)),
        out_shape=jax.ShapeDtypeStruct((N, C), jnp.float32),
    )(G2, dists, params)


def _prop345_conf(G3, dists, msg1, msg2, params):
    return pl.pallas_call(
        _prop345_conf_body,
        grid=(N // _MB,),
        in_specs=[
            pl.BlockSpec((_MB, C * K32), lambda i: (i, 0)),
            pl.BlockSpec((_MB, K32), lambda i: (i, 0)),
            pl.BlockSpec((_MB, C), lambda i: (i, 0)),
            pl.BlockSpec((_MB, C), lambda i: (i, 0)),
            pl.BlockSpec((8, 128), lambda i: (0, 0)),
        ],
        out_specs=pl.BlockSpec((_MB, 1), lambda i: (i, 0)),
        out_shape=jax.ShapeDtypeStruct((N, 1), jnp.float32),
    )(G3, dists, msg1, msg2, params)


_BCOLS = 10112
_QROWS = 200


def _knn_body(q_ref, b_ref, idx_ref, dv_ref, d_ref):
    from jax.experimental.pallas import tpu as pltpu
    q = q_ref[...]
    qx, qy, qz = q[:, 0:1], q[:, 1:2], q[:, 2:3]
    bx = b_ref[0:1, :]
    by = b_ref[1:2, :]
    bz = b_ref[2:3, :]
    d_ref[...] = (qx - bx) ** 2 + (qy - by) ** 2 + (qz - bz) ** 2
    col = jax.lax.broadcasted_iota(jnp.int32, (_QROWS, _BCOLS), 1)
    lane = jax.lax.broadcasted_iota(jnp.int32, (_QROWS, K32), 1)

    def body(t, acc):
        acc_idx, acc_dv = acc
        d = d_ref[...]
        dmin = jnp.min(d, axis=1, keepdims=True)
        eq = d == dmin
        idx = jnp.min(jnp.where(eq, col, jnp.int32(2 ** 30)),
                      axis=1, keepdims=True)
        d_ref[...] = jnp.where(eq, jnp.float32(3e38), d)
        acc_idx = jnp.where(lane == t, idx, acc_idx)
        acc_dv = jnp.where(lane == t, dmin, acc_dv)
        return acc_idx, acc_dv

    acc_idx = jnp.zeros((_QROWS, K32), jnp.int32)
    acc_dv = jnp.zeros((_QROWS, K32), jnp.float32)
    acc_idx, acc_dv = jax.lax.fori_loop(0, K32, body, (acc_idx, acc_dv))
    idx_ref[...] = acc_idx
    dv_ref[...] = acc_dv


def _knn32(pos):
    from jax.experimental.pallas import tpu as pltpu
    posT = jnp.pad(pos, ((0, _BCOLS - N), (0, 0)),
                   constant_values=1e4).T
    idx, dv = pl.pallas_call(
        _knn_body,
        grid=(N // _QROWS,),
        in_specs=[
            pl.BlockSpec((_QROWS, 3), lambda i: (i, 0)),
            pl.BlockSpec((3, _BCOLS), lambda i: (0, 0)),
        ],
        out_specs=[
            pl.BlockSpec((_QROWS, K32), lambda i: (i, 0)),
            pl.BlockSpec((_QROWS, K32), lambda i: (i, 0)),
        ],
        out_shape=[
            jax.ShapeDtypeStruct((N, K32), jnp.int32),
            jax.ShapeDtypeStruct((N, K32), jnp.float32),
        ],
        scratch_shapes=[pltpu.VMEM((_QROWS, _BCOLS), jnp.float32)],
    )(pos, posT)
    return idx, dv


def _fps_select(pos6, m):
    n = pos6.shape[0]
    sel = jnp.zeros((m,), dtype=jnp.int32)
    dist = jnp.full((n,), jnp.inf, dtype=pos6.dtype)
    def body(i, state):
        dist, sel = state
        d = jnp.sum((pos6 - pos6[sel[i]]) ** 2, axis=-1)
        dist = jnp.minimum(dist, d)
        sel = sel.at[i + 1].set(jnp.argmax(dist).astype(jnp.int32))
        return (dist, sel)
    dist, sel = jax.lax.fori_loop(0, m - 1, body, (dist, sel))
    return sel


_NPAD = 10240
_RT = _NPAD // 128


def _rigid_body(Pt_ref, Qt_ref, R_ref, t_ref):
    P = [Pt_ref[j] for j in range(30)]
    Q = [Qt_ref[j] for j in range(30)]
    pc = [sum(P[3 * k + i] for k in range(K10)) * (1.0 / K10) for i in range(3)]
    qc = [sum(Q[3 * k + i] for k in range(K10)) * (1.0 / K10) for i in range(3)]
    S = [[None] * 3 for _ in range(3)]
    for i in range(3):
        for j in range(3):
            acc = None
            for k in range(K10):
                term = (P[3 * k + i] - pc[i]) * (Q[3 * k + j] - qc[j])
                acc = term if acc is None else acc + term
            S[i][j] = acc
    Sxx, Sxy, Sxz = S[0]
    Syx, Syy, Syz = S[1]
    Szx, Szy, Szz = S[2]
    b00 = Sxx + Syy + Szz
    b01 = Syz - Szy
    b02 = Szx - Sxz
    b03 = Sxy - Syx
    b11 = Sxx - Syy - Szz
    b12 = Sxy + Syx
    b13 = Szx + Sxz
    b22 = -Sxx + Syy - Szz
    b23 = Syz + Szy
    b33 = -Sxx - Syy + Szz
    fro = jnp.sqrt(b00 * b00 + b11 * b11 + b22 * b22 + b33 * b33
                   + 2.0 * (b01 * b01 + b02 * b02 + b03 * b03
                            + b12 * b12 + b13 * b13 + b23 * b23)) + 1e-30
    b00 = b00 + fro
    b11 = b11 + fro
    b22 = b22 + fro
    b33 = b33 + fro
    for _ in range(24):
        n00 = b00 * b00 + b01 * b01 + b02 * b02 + b03 * b03
        n01 = b00 * b01 + b01 * b11 + b02 * b12 + b03 * b13
        n02 = b00 * b02 + b01 * b12 + b02 * b22 + b03 * b23
        n03 = b00 * b03 + b01 * b13 + b02 * b23 + b03 * b33
        n11 = b01 * b01 + b11 * b11 + b12 * b12 + b13 * b13
        n12 = b01 * b02 + b11 * b12 + b12 * b22 + b13 * b23
        n13 = b01 * b03 + b11 * b13 + b12 * b23 + b13 * b33
        n22 = b02 * b02 + b12 * b12 + b22 * b22 + b23 * b23
        n23 = b02 * b03 + b12 * b13 + b22 * b23 + b23 * b33
        n33 = b03 * b03 + b13 * b13 + b23 * b23 + b33 * b33
        nrm = jax.lax.rsqrt(n00 * n00 + n11 * n11 + n22 * n22 + n33 * n33
                            + 2.0 * (n01 * n01 + n02 * n02 + n03 * n03
                                     + n12 * n12 + n13 * n13 + n23 * n23)
                            + 1e-38)
        b00 = n00 * nrm
        b01 = n01 * nrm
        b02 = n02 * nrm
        b03 = n03 * nrm
        b11 = n11 * nrm
        b12 = n12 * nrm
        b13 = n13 * nrm
        b22 = n22 * nrm
        b23 = n23 * nrm
        b33 = n33 * nrm
    c0 = jnp.logical_and(jnp.logical_and(b00 >= b11, b00 >= b22), b00 >= b33)
    c1 = jnp.logical_and(b11 >= b22, b11 >= b33)
    c2 = b22 >= b33
    def pick(v0, v1, v2, v3):
        return jnp.where(c0, v0, jnp.where(c1, v1, jnp.where(c2, v2, v3)))
    qw = pick(b00, b01, b02, b03)
    qx = pick(b01, b11, b12, b13)
    qy = pick(b02, b12, b22, b23)
    qz = pick(b03, b13, b23, b33)
    qn = jax.lax.rsqrt(qw * qw + qx * qx + qy * qy + qz * qz + 1e-38)
    qw, qx, qy, qz = qw * qn, qx * qn, qy * qn, qz * qn
    r00 = 1.0 - 2.0 * (qy * qy + qz * qz)
    r01 = 2.0 * (qx * qy - qw * qz)
    r02 = 2.0 * (qx * qz + qw * qy)
    r10 = 2.0 * (qx * qy + qw * qz)
    r11 = 1.0 - 2.0 * (qx * qx + qz * qz)
    r12 = 2.0 * (qy * qz - qw * qx)
    r20 = 2.0 * (qx * qz - qw * qy)
    r21 = 2.0 * (qy * qz + qw * qx)
    r22 = 1.0 - 2.0 * (qx * qx + qy * qy)
    R = [r00, r01, r02, r10, r11, r12, r20, r21, r22]
    for j in range(9):
        R_ref[j] = R[j]
    t_ref[0] = qc[0] - (r00 * pc[0] + r01 * pc[1] + r02 * pc[2])
    t_ref[1] = qc[1] - (r10 * pc[0] + r11 * pc[1] + r12 * pc[2])
    t_ref[2] = qc[2] - (r20 * pc[0] + r21 * pc[1] + r22 * pc[2])


def _rigid_fitting(pos, x, nbr):
    P = pos[nbr].reshape(N, 30)
    Q = x[nbr].reshape(N, 30)
    pad = ((0, _NPAD - N), (0, 0))
    Pt = jnp.pad(P, pad).T.reshape(30, _RT, 128)
    Qt = jnp.pad(Q, pad).T.reshape(30, _RT, 128)
    R9, t3 = pl.pallas_call(
        _rigid_body,
        in_specs=[
            pl.BlockSpec((30, _RT, 128), lambda: (0, 0, 0)),
            pl.BlockSpec((30, _RT, 128), lambda: (0, 0, 0)),
        ],
        out_specs=[
            pl.BlockSpec((9, _RT, 128), lambda: (0, 0, 0)),
            pl.BlockSpec((3, _RT, 128), lambda: (0, 0, 0)),
        ],
        out_shape=[
            jax.ShapeDtypeStruct((9, _RT, 128), jnp.float32),
            jax.ShapeDtypeStruct((3, _RT, 128), jnp.float32),
        ],
    )(Pt, Qt)
    R = R9.reshape(9, _NPAD).T[:N].reshape(N, 3, 3)
    t = t3.reshape(3, _NPAD).T[:N]
    return R, t


def kernel(x0, pos0, batch0, lin_w, lin_b, pred_w, pred_b):
    n = N
    ar = jnp.arange(n)

    nbr32, d32 = _knn32(pos0)
    nbr10 = nbr32[:, :K10]

    R0, t0 = _rigid_fitting(pos0, x0, nbr10)

    pos6d = jnp.concatenate([pos0, x0], axis=-1)
    sel = _fps_select(pos6d, C)
    cent = pos6d[sel]
    dcl = jnp.sum((pos6d[:, None, :] - cent[None, :, :]) ** 2, axis=-1)
    cl = jnp.argmin(dcl, axis=1).astype(jnp.int32)

    srcr = jnp.where(d32 <= RADIUS ** 2, nbr32, ar[:, None].astype(nbr32.dtype))
    srcf = srcr.reshape(-1)

    params = (jnp.zeros((8, 128), jnp.float32)
              .at[0, :5].set(lin_w)
              .at[1, :5].set(lin_b)
              .at[2, :5].set(pred_w[0])
              .at[3, 0].set(pred_b[0]))
    R9 = R0.reshape(n, 9)

    T1 = jnp.concatenate([pos0, x0, cl[:, None].astype(jnp.float32),
                          jnp.zeros((n, 9), jnp.float32)], axis=1)
    G1 = _sc_gather(T1, srcf, 128).reshape(n, 16 * K32)
    dists, msg1 = _edge_stage(G1, R9, t0, params)

    G2 = _sc_gather(msg1, srcf, 128).reshape(n, C * K32)
    msg2 = _prop2(G2, dists, params)
    G3 = _sc_gather(msg2, srcf, 128).reshape(n, C * K32)
    conf0 = _prop345_conf(G3, dists, msg1, msg2, params)

    T4 = jnp.concatenate([conf0, R9, t0,
                          jnp.zeros((n, 3), jnp.float32)], axis=1)
    G4 = _sc_gather(T4, nbr10.reshape(-1), 160).reshape(n, 16 * K10)
    x_out, R_out9, t_out = _finalize(G4, pos0)
    return (x_out, R_out9.reshape(n, 3, 3), t_out)
